# baseline (speedup 1.0000x reference)
import jax
import jax.numpy as jnp
from jax.experimental import pallas as pl
from jax.experimental.pallas import tpu as pltpu

_B = 2
_CIN = 256
_FIL = 256
_A = 9
_H = 48
_W = 48
_HW = _H * _W
_N = _HW * _A
_ROWS = _N // 128
_MAX_OUT = 300
_OUT_ROWS = 304
_NMS_T = 0.7
_G = 4


def _conv_head_body(xsh_ref, w1_ref, b1_ref, wh_ref, bh_ref, out_ref):
    acc = None
    for kh in range(3):
        for kw in range(3):
            xs = xsh_ref[0, kw, kh:kh + _H, :, :].reshape(_HW, _CIN)
            t = kh * 3 + kw
            p = jax.lax.dot_general(
                xs, w1_ref[t * _CIN:(t + 1) * _CIN, :], (((1,), (0,)), ((), ())),
                preferred_element_type=jnp.float32)
            acc = p if acc is None else acc + p
    l = jax.nn.relu(acc + b1_ref[...][None, :])
    out_ref[0] = jax.lax.dot_general(
        l, wh_ref[...], (((1,), (0,)), ((), ())),
        preferred_element_type=jnp.float32) + bh_ref[...][None, :]


def _nms_body(c0_ref, c1_ref, dx_ref, dy_ref, dw_ref, dh_ref,
              ax1_ref, ay1_ref, ax2_ref, ay2_ref, out_ref,
              x1_s, y1_s, x2_s, y2_s, ar_s, cur_s):
    f0 = jnp.float32(0.0)
    ninf = jnp.float32(-jnp.inf)
    T = jnp.float32(_NMS_T)
    BIGI = jnp.int32(2**31 - 1)

    ax1 = ax1_ref[0]; ay1 = ay1_ref[0]; ax2 = ax2_ref[0]; ay2 = ay2_ref[0]
    widths = ax2 - ax1 + 1.0
    heights = ay2 - ay1 + 1.0
    cx = ax1 + 0.5 * widths
    cy = ay1 + 0.5 * heights
    dx = dx_ref[...]; dy = dy_ref[...]; dw = dw_ref[...]; dh = dh_ref[...]
    pcx = dx * widths + cx
    pcy = dy * heights + cy
    pw = jnp.exp(dw) * widths
    ph = jnp.exp(dh) * heights
    x1 = pcx - 0.5 * pw
    y1 = pcy - 0.5 * ph
    x2 = pcx + 0.5 * pw
    y2 = pcy + 0.5 * ph
    x1_s[...] = x1
    y1_s[...] = y1
    x2_s[...] = x2
    y2_s[...] = y2
    ar_s[...] = (x2 - x1) * (y2 - y1)

    c0 = c0_ref[...]; c1 = c1_ref[...]
    m = jnp.maximum(c0, c1)
    e0 = jnp.exp(c0 - m)
    e1 = jnp.exp(c1 - m)
    s = e0 + e1
    sm0 = e0 / s
    sm1 = e1 / s
    sc = jnp.maximum(sm0, sm1)
    cur_s[...] = jnp.where(sm1 > sm0, sc, ninf)

    lane3 = jax.lax.broadcasted_iota(jnp.int32, (_B, _OUT_ROWS, 128), 2)
    out_ref[...] = jnp.where(lane3 == 4, jnp.float32(-1.0), f0)

    iota_flat = (jax.lax.broadcasted_iota(jnp.int32, (_ROWS, 128), 0) * 128
                 + jax.lax.broadcasted_iota(jnp.int32, (_ROWS, 128), 1))
    lane_row = jax.lax.broadcasted_iota(jnp.int32, (1, 128), 1)

    def image_group(i, k, live):
        cur = cur_s[i]

        vals, idxs, acts = [], [], []
        curv = cur
        for j in range(_G):
            mj = jnp.max(curv)
            ij = jnp.min(jnp.where(curv == mj, iota_flat, BIGI))
            aj = mj > ninf
            ij = jnp.where(aj, ij, 0)
            if j + 1 < _G:
                curv = jnp.where(iota_flat == ij, ninf, curv)
            vals.append(mj); idxs.append(ij); acts.append(aj)

        boxes = []
        for j in range(_G):
            r = jax.lax.shift_right_logical(idxs[j], 7)
            c = jax.lax.bitwise_and(idxs[j], 127)

            def pick(ref, r=r, c=c):
                rowv = ref[i, pl.ds(r, 1), :]
                return jnp.max(jnp.where(lane_row == c, rowv, ninf))

            boxes.append((pick(x1_s), pick(y1_s), pick(x2_s), pick(y2_s), pick(ar_s)))

        x1v = x1_s[i]; y1v = y1_s[i]; x2v = x2_s[i]; y2v = y2_s[i]
        arv = ar_s[i]
        sups = []
        for j in range(_G):
            x1i, y1i, x2i, y2i, ari = boxes[j]
            xx1 = jnp.maximum(x1i, x1v)
            yy1 = jnp.maximum(y1i, y1v)
            xx2 = jnp.minimum(x2i, x2v)
            yy2 = jnp.minimum(y2i, y2v)
            w = jnp.maximum(f0, xx2 - xx1)
            h = jnp.maximum(f0, yy2 - yy1)
            inter = w * h
            iou = inter / (ari + arv - inter + jnp.float32(1e-12))
            sups.append(iou > T)

        def hit(j, kk):
            msk = jnp.where(iota_flat == idxs[kk], sups[j].astype(jnp.int32), 0)
            return jnp.max(msk) > 0

        accs = [acts[0]]
        for kk in range(1, _G):
            sup = hit(0, kk) & accs[0]
            for j in range(1, kk):
                sup = sup | (hit(j, kk) & accs[j])
            accs.append(acts[kk] & jnp.logical_not(sup))

        supA = accs[0] & sups[0]
        for j in range(1, _G):
            supA = supA | (accs[j] & sups[j])

        @pl.when(live)
        def _():
            cur_s[i] = jnp.where(supA, ninf, cur)

        rowk = k
        for j in range(_G):
            x1i, y1i, x2i, y2i, _unused = boxes[j]
            row = jnp.where(lane_row == 0, x1i,
                  jnp.where(lane_row == 1, y1i,
                  jnp.where(lane_row == 2, x2i,
                  jnp.where(lane_row == 3, y2i,
                  jnp.where(lane_row == 4, vals[j], f0)))))
            commit = live & accs[j]

            @pl.when(commit)
            def _(rowk=rowk, row=row):
                out_ref[i, pl.ds(rowk, 1), :] = row

            rowk = rowk + (live & accs[j]).astype(jnp.int32)

        live2 = live & acts[0] & (rowk < _MAX_OUT)
        return rowk, live2

    def cond(st):
        _k0, _k1, l0, l1 = st
        return l0 | l1

    def body(st):
        k0, k1, l0, l1 = st
        k0n, l0n = image_group(0, k0, l0)
        k1n, l1n = image_group(1, k1, l1)
        return (k0n, k1n, l0n, l1n)

    jax.lax.while_loop(
        cond, body,
        (jnp.int32(0), jnp.int32(0), jnp.bool_(True), jnp.bool_(True)))


@jax.jit
def kernel(feature, anchors, W1, b1, Wc, bc, Wr, br):
    f32 = jnp.float32
    feature = feature.astype(f32)

    xp = jnp.transpose(jnp.pad(feature, ((0, 0), (0, 0), (1, 1), (1, 1))),
                       (0, 2, 3, 1))
    xsh = jnp.stack([xp[:, :, kw:kw + _W, :] for kw in range(3)], axis=1)
    w1_mat = jnp.concatenate(
        [W1[:, :, kh, kw].T for kh in range(3) for kw in range(3)], axis=0)

    Wc2 = Wc[:, :, 0, 0]
    Wr2 = Wr[:, :, 0, 0]
    cols = []
    bvals = []
    for f in range(6):
        for a in range(_A):
            if f < 2:
                cols.append(Wc2[2 * a + f])
                bvals.append(bc[2 * a + f])
            else:
                cols.append(Wr2[4 * a + (f - 2)])
                bvals.append(br[4 * a + (f - 2)])
    wh = jnp.pad(jnp.stack(cols, axis=1), ((0, 0), (0, 128 - 54)))
    bh = jnp.pad(jnp.stack(bvals), (0, 128 - 54))

    head = pl.pallas_call(
        _conv_head_body,
        grid=(_B,),
        in_specs=[
            pl.BlockSpec((1, 3, 50, _W, _CIN), lambda b: (b, 0, 0, 0, 0)),
            pl.BlockSpec((9 * _CIN, _FIL), lambda b: (0, 0)),
            pl.BlockSpec((_FIL,), lambda b: (0,)),
            pl.BlockSpec((_FIL, 128), lambda b: (0, 0)),
            pl.BlockSpec((128,), lambda b: (0,)),
        ],
        out_specs=pl.BlockSpec((1, _HW, 128), lambda b: (b, 0, 0)),
        out_shape=jax.ShapeDtypeStruct((_B, _HW, 128), f32),
    )(xsh, w1_mat, b1, wh, bh)

    def plane(f):
        return head[:, :, f * _A:(f + 1) * _A].reshape(_B, _ROWS, 128)

    c0, c1, dxp, dyp, dwp, dhp = (plane(f) for f in range(6))
    ap = [anchors[:, i].reshape(_ROWS, 128) for i in range(4)]

    vec_spec = pl.BlockSpec((_B, _ROWS, 128), lambda: (0, 0, 0))
    anc_spec = pl.BlockSpec((1, _ROWS, 128), lambda: (0, 0, 0))
    scratch = [pltpu.VMEM((_B, _ROWS, 128), f32) for _ in range(6)]

    out = pl.pallas_call(
        _nms_body,
        grid=(),
        in_specs=[vec_spec] * 6 + [anc_spec] * 4,
        out_specs=pl.BlockSpec((_B, _OUT_ROWS, 128), lambda: (0, 0, 0)),
        out_shape=jax.ShapeDtypeStruct((_B, _OUT_ROWS, 128), f32),
        scratch_shapes=scratch,
    )(c0, c1, dxp, dyp, dwp, dhp,
      ap[0][None], ap[1][None], ap[2][None], ap[3][None])

    prop_b = out[:, :_MAX_OUT, 0:4]
    prop_s = out[:, :_MAX_OUT, 4]
    return (prop_b, prop_s)


# speedup vs baseline: 1.0001x; 1.0001x over previous
import jax
import jax.numpy as jnp
from jax.experimental import pallas as pl
from jax.experimental.pallas import tpu as pltpu

_B = 2
_CIN = 256
_FIL = 256
_A = 9
_H = 48
_W = 48
_HW = _H * _W
_N = _HW * _A
_ROWS = _N // 128
_MAX_OUT = 300
_OUT_ROWS = 304
_NMS_T = 0.7
_G = 4


def _conv_head_body(xsh_ref, w1_ref, b1_ref, wh_ref, bh_ref, out_ref):
    acc = None
    for kh in range(3):
        for kw in range(3):
            xs = xsh_ref[0, kw, kh:kh + _H, :, :].reshape(_HW, _CIN)
            t = kh * 3 + kw
            p = jax.lax.dot_general(
                xs, w1_ref[t * _CIN:(t + 1) * _CIN, :], (((1,), (0,)), ((), ())),
                preferred_element_type=jnp.float32)
            acc = p if acc is None else acc + p
    l = jax.nn.relu(acc + b1_ref[...][None, :])
    out_ref[0] = jax.lax.dot_general(
        l, wh_ref[...], (((1,), (0,)), ((), ())),
        preferred_element_type=jnp.float32) + bh_ref[...][None, :]


def _nms_body(c0_ref, c1_ref, dx_ref, dy_ref, dw_ref, dh_ref,
              ax1_ref, ay1_ref, ax2_ref, ay2_ref, out_ref,
              x1_s0, y1_s0, x2_s0, y2_s0, ar_s0, cur_s0,
              x1_s1, y1_s1, x2_s1, y2_s1, ar_s1, cur_s1):
    x1_s = [x1_s0, x1_s1]; y1_s = [y1_s0, y1_s1]
    x2_s = [x2_s0, x2_s1]; y2_s = [y2_s0, y2_s1]
    ar_s = [ar_s0, ar_s1]; cur_s = [cur_s0, cur_s1]
    f0 = jnp.float32(0.0)
    ninf = jnp.float32(-jnp.inf)
    T = jnp.float32(_NMS_T)
    BIGI = jnp.int32(2**31 - 1)

    ax1 = ax1_ref[0]; ay1 = ay1_ref[0]; ax2 = ax2_ref[0]; ay2 = ay2_ref[0]
    widths = ax2 - ax1 + 1.0
    heights = ay2 - ay1 + 1.0
    cx = ax1 + 0.5 * widths
    cy = ay1 + 0.5 * heights
    dx = dx_ref[...]; dy = dy_ref[...]; dw = dw_ref[...]; dh = dh_ref[...]
    pcx = dx * widths + cx
    pcy = dy * heights + cy
    pw = jnp.exp(dw) * widths
    ph = jnp.exp(dh) * heights
    x1 = pcx - 0.5 * pw
    y1 = pcy - 0.5 * ph
    x2 = pcx + 0.5 * pw
    y2 = pcy + 0.5 * ph
    c0 = c0_ref[...]; c1 = c1_ref[...]
    m = jnp.maximum(c0, c1)
    e0 = jnp.exp(c0 - m)
    e1 = jnp.exp(c1 - m)
    s = e0 + e1
    sm0 = e0 / s
    sm1 = e1 / s
    sc = jnp.maximum(sm0, sm1)
    cur0 = jnp.where(sm1 > sm0, sc, ninf)
    for i in range(_B):
        x1_s[i][...] = x1[i]
        y1_s[i][...] = y1[i]
        x2_s[i][...] = x2[i]
        y2_s[i][...] = y2[i]
        ar_s[i][...] = (x2[i] - x1[i]) * (y2[i] - y1[i])
        cur_s[i][...] = cur0[i]

    lane3 = jax.lax.broadcasted_iota(jnp.int32, (_B, _OUT_ROWS, 128), 2)
    out_ref[...] = jnp.where(lane3 == 4, jnp.float32(-1.0), f0)

    iota_flat = (jax.lax.broadcasted_iota(jnp.int32, (_ROWS, 128), 0) * 128
                 + jax.lax.broadcasted_iota(jnp.int32, (_ROWS, 128), 1))
    lane_row = jax.lax.broadcasted_iota(jnp.int32, (1, 128), 1)

    def image_group(i, k, live):
        cur = cur_s[i][...]

        vals, idxs, acts = [], [], []
        curv = cur
        for j in range(_G):
            mj = jnp.max(curv)
            ij = jnp.min(jnp.where(curv == mj, iota_flat, BIGI))
            aj = mj > ninf
            ij = jnp.where(aj, ij, 0)
            if j + 1 < _G:
                curv = jnp.where(iota_flat == ij, ninf, curv)
            vals.append(mj); idxs.append(ij); acts.append(aj)

        boxes = []
        for j in range(_G):
            r = jax.lax.shift_right_logical(idxs[j], 7)
            c = jax.lax.bitwise_and(idxs[j], 127)

            def pick(ref, r=r, c=c):
                rowv = ref[pl.ds(r, 1), :]
                return jnp.max(jnp.where(lane_row == c, rowv, ninf))

            boxes.append((pick(x1_s[i]), pick(y1_s[i]), pick(x2_s[i]),
                          pick(y2_s[i]), pick(ar_s[i])))

        x1v = x1_s[i][...]; y1v = y1_s[i][...]; x2v = x2_s[i][...]
        y2v = y2_s[i][...]; arv = ar_s[i][...]
        sups = []
        for j in range(_G):
            x1i, y1i, x2i, y2i, ari = boxes[j]
            xx1 = jnp.maximum(x1i, x1v)
            yy1 = jnp.maximum(y1i, y1v)
            xx2 = jnp.minimum(x2i, x2v)
            yy2 = jnp.minimum(y2i, y2v)
            w = jnp.maximum(f0, xx2 - xx1)
            h = jnp.maximum(f0, yy2 - yy1)
            inter = w * h
            iou = inter / (ari + arv - inter + jnp.float32(1e-12))
            sups.append(iou > T)

        def hit(j, kk):
            msk = jnp.where(iota_flat == idxs[kk], sups[j].astype(jnp.int32), 0)
            return jnp.max(msk) > 0

        accs = [acts[0]]
        for kk in range(1, _G):
            sup = hit(0, kk) & accs[0]
            for j in range(1, kk):
                sup = sup | (hit(j, kk) & accs[j])
            accs.append(acts[kk] & jnp.logical_not(sup))

        supA = accs[0] & sups[0]
        for j in range(1, _G):
            supA = supA | (accs[j] & sups[j])

        @pl.when(live)
        def _():
            cur_s[i][...] = jnp.where(supA, ninf, cur)

        rowk = k
        for j in range(_G):
            x1i, y1i, x2i, y2i, _unused = boxes[j]
            row = jnp.where(lane_row == 0, x1i,
                  jnp.where(lane_row == 1, y1i,
                  jnp.where(lane_row == 2, x2i,
                  jnp.where(lane_row == 3, y2i,
                  jnp.where(lane_row == 4, vals[j], f0)))))
            commit = live & accs[j]

            @pl.when(commit)
            def _(rowk=rowk, row=row):
                out_ref[i, pl.ds(rowk, 1), :] = row

            rowk = rowk + (live & accs[j]).astype(jnp.int32)

        live2 = live & acts[0] & (rowk < _MAX_OUT)
        return rowk, live2

    def cond(st):
        _k0, _k1, l0, l1 = st
        return l0 | l1

    def body(st):
        k0, k1, l0, l1 = st
        k0n, l0n = image_group(0, k0, l0)
        k1n, l1n = image_group(1, k1, l1)
        return (k0n, k1n, l0n, l1n)

    jax.lax.while_loop(
        cond, body,
        (jnp.int32(0), jnp.int32(0), jnp.bool_(True), jnp.bool_(True)))


@jax.jit
def kernel(feature, anchors, W1, b1, Wc, bc, Wr, br):
    f32 = jnp.float32
    feature = feature.astype(f32)

    xp = jnp.transpose(jnp.pad(feature, ((0, 0), (0, 0), (1, 1), (1, 1))),
                       (0, 2, 3, 1))
    xsh = jnp.stack([xp[:, :, kw:kw + _W, :] for kw in range(3)], axis=1)
    w1_mat = jnp.concatenate(
        [W1[:, :, kh, kw].T for kh in range(3) for kw in range(3)], axis=0)

    Wc2 = Wc[:, :, 0, 0]
    Wr2 = Wr[:, :, 0, 0]
    cols = []
    bvals = []
    for f in range(6):
        for a in range(_A):
            if f < 2:
                cols.append(Wc2[2 * a + f])
                bvals.append(bc[2 * a + f])
            else:
                cols.append(Wr2[4 * a + (f - 2)])
                bvals.append(br[4 * a + (f - 2)])
    wh = jnp.pad(jnp.stack(cols, axis=1), ((0, 0), (0, 128 - 54)))
    bh = jnp.pad(jnp.stack(bvals), (0, 128 - 54))

    head = pl.pallas_call(
        _conv_head_body,
        grid=(_B,),
        in_specs=[
            pl.BlockSpec((1, 3, 50, _W, _CIN), lambda b: (b, 0, 0, 0, 0)),
            pl.BlockSpec((9 * _CIN, _FIL), lambda b: (0, 0)),
            pl.BlockSpec((_FIL,), lambda b: (0,)),
            pl.BlockSpec((_FIL, 128), lambda b: (0, 0)),
            pl.BlockSpec((128,), lambda b: (0,)),
        ],
        out_specs=pl.BlockSpec((1, _HW, 128), lambda b: (b, 0, 0)),
        out_shape=jax.ShapeDtypeStruct((_B, _HW, 128), f32),
    )(xsh, w1_mat, b1, wh, bh)

    def plane(f):
        return head[:, :, f * _A:(f + 1) * _A].reshape(_B, _ROWS, 128)

    c0, c1, dxp, dyp, dwp, dhp = (plane(f) for f in range(6))
    ap = [anchors[:, i].reshape(_ROWS, 128) for i in range(4)]

    vec_spec = pl.BlockSpec((_B, _ROWS, 128), lambda: (0, 0, 0))
    anc_spec = pl.BlockSpec((1, _ROWS, 128), lambda: (0, 0, 0))
    scratch = [pltpu.VMEM((_ROWS, 128), f32) for _ in range(12)]

    out = pl.pallas_call(
        _nms_body,
        grid=(),
        in_specs=[vec_spec] * 6 + [anc_spec] * 4,
        out_specs=pl.BlockSpec((_B, _OUT_ROWS, 128), lambda: (0, 0, 0)),
        out_shape=jax.ShapeDtypeStruct((_B, _OUT_ROWS, 128), f32),
        scratch_shapes=scratch,
    )(c0, c1, dxp, dyp, dwp, dhp,
      ap[0][None], ap[1][None], ap[2][None], ap[3][None])

    prop_b = out[:, :_MAX_OUT, 0:4]
    prop_s = out[:, :_MAX_OUT, 4]
    return (prop_b, prop_s)
